# baseline (device time: 176450 ns/iter reference)
import jax
import jax.numpy as jnp
from jax import lax
from jax.experimental import pallas as pl
from jax.experimental.pallas import tpu as pltpu

NX, NY, NZ = 2, 4, 4
NG = NY * NZ
D = 2048
F = 8192
FB = F // NG
HALF = D // NX


_CYCLE = [
    (0, 0), (0, 1), (0, 2), (0, 3),
    (1, 3), (1, 2), (1, 1),
    (2, 1), (2, 2), (2, 3),
    (3, 3), (3, 2), (3, 1), (3, 0),
    (2, 0), (1, 0),
]
_YC = jnp.array([c[0] for c in _CYCLE], jnp.int32)
_ZC = jnp.array([c[1] for c in _CYCLE], jnp.int32)
_POS = jnp.zeros((NY, NZ), jnp.int32).at[
    tuple(zip(*_CYCLE))].set(jnp.arange(NG, dtype=jnp.int32))


def kernel(x, dy):
    m_loc = x.shape[0]

    my_y = lax.axis_index("y")
    my_z = lax.axis_index("z")
    p = _POS[my_y, my_z]
    pr = (p + 1) % NG
    pl_ = (p - 1) % NG
    meta = jnp.stack([p, _YC[pr], _ZC[pr], _YC[pl_], _ZC[pl_]]).astype(jnp.int32)

    xt_bf = x.T.astype(jnp.bfloat16)
    dy_blk = lax.dynamic_slice(dy, (0, p * FB), (m_loc, FB)).astype(jnp.bfloat16)

    def body(meta_ref, x_ref, dy_ref, out_ref, xsend_ref, xrecv_ref,
             send_sems_r, send_sems_l, recv_sems, xsend_sem, xrecv_sem):
        my_x = lax.axis_index("x")
        my_y = lax.axis_index("y")
        my_z = lax.axis_index("z")
        p = meta_ref[0]
        ry, rz = meta_ref[1], meta_ref[2]
        ly, lz = meta_ref[3], meta_ref[4]

        barrier_sem = pltpu.get_barrier_semaphore()
        for dev in [(my_x, ry, rz), (my_x, ly, lz), (1 - my_x, my_y, my_z)]:
            pl.semaphore_signal(
                barrier_sem, inc=1, device_id=dev,
                device_id_type=pl.DeviceIdType.MESH,
            )

        other_off = (1 - my_x) * HALF
        xsend_ref[...] = lax.dot_general(
            x_ref[pl.ds(other_off, HALF), :],
            dy_ref[...],
            (((1,), (0,)), ((), ())),
            preferred_element_type=jnp.float32,
        ).astype(jnp.bfloat16)

        pl.semaphore_wait(barrier_sem, 3)
        xrdma = pltpu.make_async_remote_copy(
            src_ref=xsend_ref,
            dst_ref=xrecv_ref,
            send_sem=xsend_sem,
            recv_sem=xrecv_sem,
            device_id=(1 - my_x, my_y, my_z),
            device_id_type=pl.DeviceIdType.MESH,
        )
        xrdma.start()

        my_off = my_x * HALF
        my_p = lax.dot_general(
            x_ref[pl.ds(my_off, HALF), :],
            dy_ref[...],
            (((1,), (0,)), ((), ())),
            preferred_element_type=jnp.float32,
        )
        xrdma.wait()

        s = my_p + xrecv_ref[...].astype(jnp.float32)
        out_ref[:, pl.ds(p * FB, FB)] = s.astype(jnp.bfloat16)

        H_R, H_L = NG // 2, NG - 1 - NG // 2

        def _copy(sl, dev, ssems):
            return pltpu.make_async_remote_copy(
                src_ref=out_ref.at[:, pl.ds(sl * FB, FB)],
                dst_ref=out_ref.at[:, pl.ds(sl * FB, FB)],
                send_sem=ssems.at[sl],
                recv_sem=recv_sems.at[sl],
                device_id=dev,
                device_id_type=pl.DeviceIdType.MESH,
            )

        for h in range(H_R):
            send_r = _copy((p - h) % NG, (my_x, ry, rz), send_sems_r)
            send_r.start()
            send_l = None
            if h < H_L:
                send_l = _copy((p + h) % NG, (my_x, ly, lz), send_sems_l)
                send_l.start()
            rb_r = (p - h - 1) % NG
            _copy(rb_r, (my_x, ry, rz), send_sems_r).wait_recv()
            if h < H_L:
                rb_l = (p + h + 1) % NG
                _copy(rb_l, (my_x, ly, lz), send_sems_l).wait_recv()
            send_r.wait_send()
            if send_l is not None:
                send_l.wait_send()

    return pl.pallas_call(
        body,
        out_shape=jax.ShapeDtypeStruct((HALF, F), jnp.bfloat16),
        in_specs=[
            pl.BlockSpec(memory_space=pltpu.SMEM),
            pl.BlockSpec(memory_space=pltpu.VMEM),
            pl.BlockSpec(memory_space=pltpu.VMEM),
        ],
        out_specs=pl.BlockSpec(memory_space=pltpu.VMEM),
        scratch_shapes=[
            pltpu.VMEM((HALF, FB), jnp.bfloat16),
            pltpu.VMEM((HALF, FB), jnp.bfloat16),
            pltpu.SemaphoreType.DMA((NG,)),
            pltpu.SemaphoreType.DMA((NG,)),
            pltpu.SemaphoreType.DMA((NG,)),
            pltpu.SemaphoreType.DMA,
            pltpu.SemaphoreType.DMA,
        ],
        compiler_params=pltpu.CompilerParams(
            vmem_limit_bytes=100 * 1024 * 1024,
            collective_id=0,
        ),
    )(meta, xt_bf, dy_blk)


# device time: 151184 ns/iter; 1.1671x vs baseline; 1.1671x over previous
import jax
import jax.numpy as jnp
from jax import lax
from jax.experimental import pallas as pl
from jax.experimental.pallas import tpu as pltpu

NX, NY, NZ = 2, 4, 4
NG = NY * NZ
D = 2048
F = 8192
FB = F // NG
HALF = D // NX
SUB = 2
SROWS = HALF // SUB


_CYCLE = [
    (0, 0), (0, 1), (0, 2), (0, 3),
    (1, 3), (1, 2), (1, 1),
    (2, 1), (2, 2), (2, 3),
    (3, 3), (3, 2), (3, 1), (3, 0),
    (2, 0), (1, 0),
]
_YC = jnp.array([c[0] for c in _CYCLE], jnp.int32)
_ZC = jnp.array([c[1] for c in _CYCLE], jnp.int32)
_POS = jnp.zeros((NY, NZ), jnp.int32).at[
    tuple(zip(*_CYCLE))].set(jnp.arange(NG, dtype=jnp.int32))


def kernel(x, dy):
    m_loc = x.shape[0]

    my_y = lax.axis_index("y")
    my_z = lax.axis_index("z")
    p = _POS[my_y, my_z]
    pr = (p + 1) % NG
    pl_ = (p - 1) % NG
    meta = jnp.stack([p, _YC[pr], _ZC[pr], _YC[pl_], _ZC[pl_]]).astype(jnp.int32)

    dy_blk = lax.dynamic_slice(dy, (0, p * FB), (m_loc, FB))

    def body(meta_ref, x_ref, dy_ref, out_ref, xsend_ref, xrecv_ref,
             send_sems_r, send_sems_l, recv_sems, xsend_sem, xrecv_sem):
        my_x = lax.axis_index("x")
        my_y = lax.axis_index("y")
        my_z = lax.axis_index("z")
        p = meta_ref[0]
        ry, rz = meta_ref[1], meta_ref[2]
        ly, lz = meta_ref[3], meta_ref[4]

        barrier_sem = pltpu.get_barrier_semaphore()
        for dev in [(my_x, ry, rz), (my_x, ly, lz), (1 - my_x, my_y, my_z)]:
            pl.semaphore_signal(
                barrier_sem, inc=1, device_id=dev,
                device_id_type=pl.DeviceIdType.MESH,
            )

        dy_bf = dy_ref[...].astype(jnp.bfloat16)

        other_off = (1 - my_x) * HALF
        xsend_ref[...] = lax.dot_general(
            x_ref[:, pl.ds(other_off, HALF)].astype(jnp.bfloat16),
            dy_bf,
            (((0,), (0,)), ((), ())),
            preferred_element_type=jnp.float32,
        ).astype(jnp.bfloat16)

        pl.semaphore_wait(barrier_sem, 3)
        xrdma = pltpu.make_async_remote_copy(
            src_ref=xsend_ref,
            dst_ref=xrecv_ref,
            send_sem=xsend_sem,
            recv_sem=xrecv_sem,
            device_id=(1 - my_x, my_y, my_z),
            device_id_type=pl.DeviceIdType.MESH,
        )
        xrdma.start()

        my_off = my_x * HALF
        my_p = lax.dot_general(
            x_ref[:, pl.ds(my_off, HALF)].astype(jnp.bfloat16),
            dy_bf,
            (((0,), (0,)), ((), ())),
            preferred_element_type=jnp.float32,
        )
        xrdma.wait()

        s = my_p + xrecv_ref[...].astype(jnp.float32)
        out_ref[:, pl.ds(p * FB, FB)] = s.astype(jnp.bfloat16)

        H_R, H_L = NG // 2, NG - 1 - NG // 2

        def _copy(sl, j, dev, ssems):
            blk = out_ref.at[pl.ds(j * SROWS, SROWS), pl.ds(sl * FB, FB)]
            return pltpu.make_async_remote_copy(
                src_ref=blk,
                dst_ref=blk,
                send_sem=ssems.at[sl, j],
                recv_sem=recv_sems.at[sl, j],
                device_id=dev,
                device_id_type=pl.DeviceIdType.MESH,
            )

        sends = []
        for j in range(SUB):
            sends.append(_copy(p, j, (my_x, ry, rz), send_sems_r))
            sends.append(_copy(p, j, (my_x, ly, lz), send_sems_l))
            sends[-2].start()
            sends[-1].start()
        for h in range(1, H_R + 1):
            rb_r = (p - h) % NG
            rb_l = (p + h) % NG
            for j in range(SUB):
                _copy(rb_r, j, (my_x, ry, rz), send_sems_r).wait_recv()
                if h < H_R:
                    fwd = _copy(rb_r, j, (my_x, ry, rz), send_sems_r)
                    fwd.start()
                    sends.append(fwd)
                if h <= H_L:
                    _copy(rb_l, j, (my_x, ly, lz), send_sems_l).wait_recv()
                    if h < H_L:
                        fwd = _copy(rb_l, j, (my_x, ly, lz), send_sems_l)
                        fwd.start()
                        sends.append(fwd)
        for sd in sends:
            sd.wait_send()

    return pl.pallas_call(
        body,
        out_shape=jax.ShapeDtypeStruct((HALF, F), jnp.bfloat16),
        in_specs=[
            pl.BlockSpec(memory_space=pltpu.SMEM),
            pl.BlockSpec(memory_space=pltpu.VMEM),
            pl.BlockSpec(memory_space=pltpu.VMEM),
        ],
        out_specs=pl.BlockSpec(memory_space=pltpu.VMEM),
        scratch_shapes=[
            pltpu.VMEM((HALF, FB), jnp.bfloat16),
            pltpu.VMEM((HALF, FB), jnp.bfloat16),
            pltpu.SemaphoreType.DMA((NG, SUB)),
            pltpu.SemaphoreType.DMA((NG, SUB)),
            pltpu.SemaphoreType.DMA((NG, SUB)),
            pltpu.SemaphoreType.DMA,
            pltpu.SemaphoreType.DMA,
        ],
        compiler_params=pltpu.CompilerParams(
            vmem_limit_bytes=100 * 1024 * 1024,
            collective_id=0,
        ),
    )(meta, x, dy_blk)
